# baseline (speedup 1.0000x reference)
import functools

import jax
import jax.numpy as jnp
from jax import lax
from jax.experimental import pallas as pl
from jax.experimental.pallas import tpu as pltpu

_EPS = 1e-6
_G = 32
_HW = 49
_L = _G * _HW


def _gem_body(p_ref, x_ref, seg_ref, o_ref):
    p = p_ref[0]
    xc = jnp.maximum(x_ref[...], _EPS)
    xp = jnp.exp2(jnp.log2(xc) * p).astype(jnp.bfloat16)
    s = jnp.dot(xp, seg_ref[...], preferred_element_type=jnp.float32)
    m = s * (1.0 / _HW)
    o_ref[...] = jnp.exp2(jnp.log2(m) * (1.0 / p))


def _seg_matrix():
    j = lax.broadcasted_iota(jnp.int32, (_L, _G), 0)
    c = lax.broadcasted_iota(jnp.int32, (_L, _G), 1)
    d = j - c * _HW
    return ((d >= 0) & (d < _HW)).astype(jnp.bfloat16)


@functools.partial(jax.jit, static_argnames=("row_tile",))
def _gem_pool(x, p, row_tile=512):
    N, C, H, W = x.shape
    rows = (N * C) // _G
    x2 = x.reshape(rows, _L)
    p_arr = jnp.asarray(p, jnp.float32).reshape(1)
    seg = _seg_matrix()
    nrb = rows // row_tile

    out = pl.pallas_call(
        _gem_body,
        out_shape=jax.ShapeDtypeStruct((rows, _G), jnp.float32),
        grid=(nrb,),
        in_specs=[
            pl.BlockSpec(memory_space=pltpu.SMEM),
            pl.BlockSpec((row_tile, _L), lambda i: (i, 0)),
            pl.BlockSpec((_L, _G), lambda i: (0, 0)),
        ],
        out_specs=pl.BlockSpec((row_tile, _G), lambda i: (i, 0)),
        compiler_params=pltpu.CompilerParams(
            dimension_semantics=("parallel",),
            vmem_limit_bytes=60 << 20,
        ),
        cost_estimate=pl.CostEstimate(
            flops=int(6 * N * C * H * W),
            transcendentals=int(2 * N * C * H * W),
            bytes_accessed=int(x.size * 4 + N * C * 4),
        ),
    )(p_arr, x2, seg)

    return out.reshape(N, C, 1, 1)


def kernel(x, p):
    return _gem_pool(x, p)


# speedup vs baseline: 24.1664x; 24.1664x over previous
import functools

import jax
import jax.numpy as jnp
from jax.experimental import pallas as pl
from jax.experimental.pallas import tpu as pltpu

_EPS = 1e-6


def _gem_body(p_ref, x_ref, o_ref):
    p = p_ref[0]
    xc = jnp.maximum(x_ref[...], _EPS)
    xp = jnp.exp2(jnp.log2(xc) * p)
    s = jnp.sum(xp, axis=0)
    m = s * (1.0 / x_ref.shape[0])
    o_ref[...] = jnp.exp2(jnp.log2(m) * (1.0 / p))


@jax.jit
def _gem_pool(x, p):
    N, C, H, W = x.shape
    HW = H * W
    xt = x.transpose(2, 3, 0, 1).reshape(HW, N, C)
    p_arr = jnp.asarray(p, jnp.float32).reshape(1)

    bn = 8
    grid = (N // bn,)

    out = pl.pallas_call(
        _gem_body,
        out_shape=jax.ShapeDtypeStruct((N, C), jnp.float32),
        grid=grid,
        in_specs=[
            pl.BlockSpec(memory_space=pltpu.SMEM),
            pl.BlockSpec((HW, bn, C), lambda i: (0, i, 0)),
        ],
        out_specs=pl.BlockSpec((bn, C), lambda i: (i, 0)),
        compiler_params=pltpu.CompilerParams(
            dimension_semantics=("parallel",),
            vmem_limit_bytes=60 << 20,
        ),
        cost_estimate=pl.CostEstimate(
            flops=int(2 * N * C * HW),
            transcendentals=int(2 * N * C * HW + 2 * N * C),
            bytes_accessed=int(x.size * 4 + N * C * 4),
        ),
    )(p_arr, xt)

    return out.reshape(N, C, 1, 1)


def kernel(x, p):
    return _gem_pool(x, p)


# speedup vs baseline: 27.1488x; 1.1234x over previous
import functools

import jax
import jax.numpy as jnp
from jax.experimental import pallas as pl
from jax.experimental.pallas import tpu as pltpu

_EPS = 1e-6


def _gem_body(p_ref, x_ref, o_ref):
    p = p_ref[0]
    hw = x_ref.shape[0]

    def _pow_slab(j):
        return jnp.exp2(jnp.log2(jnp.maximum(x_ref[j], _EPS)) * p)

    acc = _pow_slab(0)
    for j in range(1, hw):
        acc = acc + _pow_slab(j)
    m = acc * (1.0 / hw)
    o_ref[...] = jnp.exp2(jnp.log2(m) * (1.0 / p))


@jax.jit
def _gem_pool(x, p):
    N, C, H, W = x.shape
    HW = H * W
    xt = x.transpose(2, 3, 0, 1).reshape(HW, N, C)
    p_arr = jnp.asarray(p, jnp.float32).reshape(1)

    bn = 8
    grid = (N // bn,)

    out = pl.pallas_call(
        _gem_body,
        out_shape=jax.ShapeDtypeStruct((N, C), jnp.float32),
        grid=grid,
        in_specs=[
            pl.BlockSpec(memory_space=pltpu.SMEM),
            pl.BlockSpec((HW, bn, C), lambda i: (0, i, 0)),
        ],
        out_specs=pl.BlockSpec((bn, C), lambda i: (i, 0)),
        compiler_params=pltpu.CompilerParams(
            dimension_semantics=("parallel",),
            vmem_limit_bytes=60 << 20,
        ),
        cost_estimate=pl.CostEstimate(
            flops=int(2 * N * C * HW),
            transcendentals=int(2 * N * C * HW + 2 * N * C),
            bytes_accessed=int(x.size * 4 + N * C * 4),
        ),
    )(p_arr, xt)

    return out.reshape(N, C, 1, 1)


def kernel(x, p):
    return _gem_pool(x, p)


# speedup vs baseline: 30.4229x; 1.1206x over previous
import functools

import jax
import jax.numpy as jnp
from jax.experimental import pallas as pl
from jax.experimental.pallas import tpu as pltpu

_EPS = 1e-6


def _gem_body(p_ref, x_ref, o_ref):
    p = p_ref[0]
    hw = x_ref.shape[0]

    def _pow_slab(j):
        return jnp.exp2(jnp.log2(jnp.maximum(x_ref[j], _EPS)) * p)

    acc = _pow_slab(0)
    for j in range(1, hw):
        acc = acc + _pow_slab(j)
    m = acc * (1.0 / hw)
    o_ref[...] = jnp.exp2(jnp.log2(m) * (1.0 / p))


@jax.jit
def _gem_pool(x, p):
    N, C, H, W = x.shape
    HW = H * W
    xt = x.transpose(2, 3, 0, 1).reshape(HW, N, C)
    p_arr = jnp.asarray(p, jnp.float32).reshape(1)

    bn = 16
    grid = (N // bn,)

    out = pl.pallas_call(
        _gem_body,
        out_shape=jax.ShapeDtypeStruct((N, C), jnp.float32),
        grid=grid,
        in_specs=[
            pl.BlockSpec(memory_space=pltpu.SMEM),
            pl.BlockSpec((HW, bn, C), lambda i: (0, i, 0)),
        ],
        out_specs=pl.BlockSpec((bn, C), lambda i: (i, 0)),
        compiler_params=pltpu.CompilerParams(
            dimension_semantics=("parallel",),
            vmem_limit_bytes=60 << 20,
        ),
        cost_estimate=pl.CostEstimate(
            flops=int(2 * N * C * HW),
            transcendentals=int(2 * N * C * HW + 2 * N * C),
            bytes_accessed=int(x.size * 4 + N * C * 4),
        ),
    )(p_arr, xt)

    return out.reshape(N, C, 1, 1)


def kernel(x, p):
    return _gem_pool(x, p)
